# baseline (speedup 1.0000x reference)
import jax
import jax.numpy as jnp
from jax.experimental import pallas as pl
from jax.experimental.pallas import tpu as pltpu

_B, _N, _MA, _MB = 4, 8192, 256, 128
_CA, _CB, _CG = 256, 256, 1024
_CH = 1024
_NB = _N // _CH
_F32 = jnp.float32


def _relu(x):
    return jnp.maximum(x, 0.0)


def _mm(a, b):
    return jax.lax.dot_general(a, b, (((1,), (0,)), ((), ())),
                               preferred_element_type=_F32)


def _dgT0(a, b):
    return jax.lax.dot_general(a, b, (((0,), (0,)), ((), ())),
                               preferred_element_type=_F32)


def _dgT1(a, b):
    return jax.lax.dot_general(a, b, (((1,), (1,)), ((), ())),
                               preferred_element_type=_F32)


def _eye(n):
    r = jax.lax.broadcasted_iota(jnp.int32, (n, n), 0)
    c = jax.lax.broadcasted_iota(jnp.int32, (n, n), 1)
    return (r == c).astype(_F32)


def _segmax_update(acc_ref, fpm_ref, mki_ref, width):
    def grp(g, _):
        f8 = fpm_ref[pl.ds(g * 8, 8), :][:, None, :]
        i8 = mki_ref[0, pl.ds(g * 8, 8), 0:1][:, :, None]
        for t in range(_MA // 32):
            it3 = jax.lax.broadcasted_iota(jnp.int32, (1, 32, 1), 1) + t * 32
            sel = jnp.where(i8 == it3, f8, 0.0)
            blk = acc_ref[t * 32:(t + 1) * 32, :]
            acc_ref[t * 32:(t + 1) * 32, :] = jnp.maximum(blk, jnp.max(sel, axis=0))
        return 0
    jax.lax.fori_loop(0, _CH // 8, grp, 0, unroll=False)


def _assign_body(pc_ref, na_ref, mki_ref, cm_ref, acc_ref):
    j = pl.program_id(1)
    pcb = pc_ref[0]
    na = na_ref[0]
    pdot = _dgT0(pcb, na)
    p2 = _dgT0(pcb * pcb, jnp.ones((3, 1), _F32))
    a2 = jnp.sum(na * na, axis=0, keepdims=True)
    d2 = p2 + a2 - 2.0 * pdot
    iota = jax.lax.broadcasted_iota(jnp.int32, (_CH, _MA), 1)
    cols = []
    d = d2
    for _ in range(3):
        v = jnp.min(d, axis=1, keepdims=True)
        i = jnp.min(jnp.where(d == v, iota, _MA), axis=1, keepdims=True)
        cols.append(i)
        d = jnp.where(iota == i, jnp.float32(jnp.inf), d)
    mki_ref[0] = jnp.concatenate(cols, axis=1)
    oh = (iota == cols[0]).astype(_F32)

    @pl.when(j == 0)
    def _():
        acc_ref[...] = jnp.zeros_like(acc_ref)

    acc_ref[0:3, :] += _mm(pcb, oh)
    acc_ref[3:4, :] += jnp.sum(oh, axis=0, keepdims=True)

    @pl.when(j == _NB - 1)
    def _():
        cm_ref[0] = acc_ref[0:3, :] / (acc_ref[3:4, :] + 1e-05)


def _pn1_body(pc_ref, it_ref, sn_ref, mki_ref, cm_ref,
              w1_ref, b1_ref, w2_ref, b2_ref, w3_ref, b3_ref,
              ctr_ref, first_ref, fmax_ref, acc_ref, hpm_ref):
    j = pl.program_id(1)
    idc = mki_ref[0, :, 0:1]
    iota = jax.lax.broadcasted_iota(jnp.int32, (_CH, _MA), 1)
    oh = (iota == idc).astype(_F32)
    centers = _dgT1(cm_ref[0], oh)
    ctr_ref[0] = centers
    aug = jnp.concatenate([pc_ref[0] - centers, it_ref[0], sn_ref[0]], axis=0)
    h = _relu(_mm(w1_ref[...], aug) + b1_ref[...])
    h = _relu(_mm(w2_ref[...], h) + b2_ref[...])
    h = _relu(_mm(w3_ref[...], h) + b3_ref[...])
    first_ref[0] = h
    hpm_ref[...] = _dgT0(h, _eye(_CA // 2))

    @pl.when(j == 0)
    def _():
        acc_ref[...] = jnp.zeros_like(acc_ref)

    _segmax_update(acc_ref, hpm_ref, mki_ref, _CA // 2)

    @pl.when(j == _NB - 1)
    def _():
        fmax_ref[0] = acc_ref[...]


def _pn2_body(first_ref, mki_ref, fmax_ref,
              w1_ref, b1_ref, w2_ref, b2_ref,
              second_ref, naf_ref, acc_ref, hpm_ref):
    j = pl.program_id(1)
    idc = mki_ref[0, :, 0:1]
    iota = jax.lax.broadcasted_iota(jnp.int32, (_CH, _MA), 1)
    oh = (iota == idc).astype(_F32)
    scattered = _mm(oh, fmax_ref[0])
    first_pm = _dgT0(first_ref[0], _eye(_CA // 2))
    fusion = jnp.concatenate([first_pm, scattered], axis=1)
    h = _relu(_dgT1(fusion, w1_ref[...]) + b1_ref[...])
    h = _relu(_dgT1(h, w2_ref[...]) + b2_ref[...])
    second_ref[0] = _dgT1(_eye(_CA), h)
    hpm_ref[...] = h

    @pl.when(j == 0)
    def _():
        acc_ref[...] = jnp.zeros_like(acc_ref)

    _segmax_update(acc_ref, hpm_ref, mki_ref, _CA)

    @pl.when(j == _NB - 1)
    def _():
        naf_ref[0] = _dgT1(_eye(_CA), acc_ref[...])


def _knn_body(nb_ref, cm_ref, naf_ref,
              k1w1_ref, k1b1_ref, k1w2_ref, k1b2_ref,
              k2w1_ref, k2b1_ref, k2w2_ref, k2b2_ref,
              fw1_ref, fb1_ref, fw2_ref, fb2_ref,
              nbf_ref, gf_ref):
    nb = nb_ref[0]
    cm = cm_ref[0]
    naf = naf_ref[0]
    qdot = _dgT0(nb, cm)
    q2 = _dgT0(nb * nb, jnp.ones((3, 1), _F32))
    c2 = jnp.sum(cm * cm, axis=0, keepdims=True)
    d2 = q2 + c2 - 2.0 * qdot
    iota = jax.lax.broadcasted_iota(jnp.int32, (_MB, _MA), 1)
    blocks = []
    d = d2
    for _ in range(16):
        v = jnp.min(d, axis=1, keepdims=True)
        i = jnp.min(jnp.where(d == v, iota, _MA), axis=1, keepdims=True)
        d = jnp.where(iota == i, jnp.float32(jnp.inf), d)
        ohk = (iota == i).astype(_F32)
        coords_k = _dgT1(cm, ohk)
        feats_k = _dgT1(naf, ohk)
        blocks.append(jnp.concatenate([coords_k - nb, feats_k], axis=0))
    y_in = jnp.concatenate(blocks, axis=1)
    h = _relu(_mm(k1w1_ref[...], y_in) + k1b1_ref[...])
    y = _relu(_mm(k1w2_ref[...], h) + k1b2_ref[...])
    ymax = y[:, 0:_MB]
    for k in range(1, 16):
        ymax = jnp.maximum(ymax, y[:, k * _MB:(k + 1) * _MB])
    y2_in = jnp.concatenate([y, jnp.concatenate([ymax] * 16, axis=1)], axis=0)
    h2 = _relu(_mm(k2w1_ref[...], y2_in) + k2b1_ref[...])
    y2 = _relu(_mm(k2w2_ref[...], h2) + k2b2_ref[...])
    nbf = y2[:, 0:_MB]
    for k in range(1, 16):
        nbf = jnp.maximum(nbf, y2[:, k * _MB:(k + 1) * _MB])
    nbf_ref[0] = nbf
    fin = jnp.concatenate([nb, nbf], axis=0)
    g1 = _relu(_mm(fw1_ref[...], fin) + fb1_ref[...])
    g2 = _relu(_mm(fw2_ref[...], g1) + fb2_ref[...])
    gf_ref[0] = jnp.max(g2, axis=1, keepdims=True)


def _col(b):
    return b.reshape(-1, 1)


def kernel(pc, intensity, sn, node_a, node_b, params):
    f32 = _F32

    grid1 = (_B, _NB)
    mki, cluster_mean = pl.pallas_call(
        _assign_body,
        grid=grid1,
        in_specs=[
            pl.BlockSpec((1, 3, _CH), lambda b, j: (b, 0, j)),
            pl.BlockSpec((1, 3, _MA), lambda b, j: (b, 0, 0)),
        ],
        out_specs=[
            pl.BlockSpec((1, _CH, 3), lambda b, j: (b, j, 0)),
            pl.BlockSpec((1, 3, _MA), lambda b, j: (b, 0, 0)),
        ],
        out_shape=[
            jax.ShapeDtypeStruct((_B, _N, 3), jnp.int32),
            jax.ShapeDtypeStruct((_B, 3, _MA), f32),
        ],
        scratch_shapes=[pltpu.VMEM((8, _MA), f32)],
        compiler_params=pltpu.CompilerParams(
            dimension_semantics=("arbitrary", "arbitrary")),
    )(pc, node_a)

    (w1, b1), (w2, b2), (w3, b3) = params['pn1']
    wspec = lambda shape: pl.BlockSpec(shape, lambda b, j: tuple(0 for _ in shape))
    pc_centers, first, fmax_pm = pl.pallas_call(
        _pn1_body,
        grid=grid1,
        in_specs=[
            pl.BlockSpec((1, 3, _CH), lambda b, j: (b, 0, j)),
            pl.BlockSpec((1, 1, _CH), lambda b, j: (b, 0, j)),
            pl.BlockSpec((1, 3, _CH), lambda b, j: (b, 0, j)),
            pl.BlockSpec((1, _CH, 3), lambda b, j: (b, j, 0)),
            pl.BlockSpec((1, 3, _MA), lambda b, j: (b, 0, 0)),
            wspec(w1.shape), wspec((w1.shape[0], 1)),
            wspec(w2.shape), wspec((w2.shape[0], 1)),
            wspec(w3.shape), wspec((w3.shape[0], 1)),
        ],
        out_specs=[
            pl.BlockSpec((1, 3, _CH), lambda b, j: (b, 0, j)),
            pl.BlockSpec((1, _CA // 2, _CH), lambda b, j: (b, 0, j)),
            pl.BlockSpec((1, _MA, _CA // 2), lambda b, j: (b, 0, 0)),
        ],
        out_shape=[
            jax.ShapeDtypeStruct((_B, 3, _N), f32),
            jax.ShapeDtypeStruct((_B, _CA // 2, _N), f32),
            jax.ShapeDtypeStruct((_B, _MA, _CA // 2), f32),
        ],
        scratch_shapes=[pltpu.VMEM((_MA, _CA // 2), f32),
                        pltpu.VMEM((_CH, _CA // 2), f32)],
        compiler_params=pltpu.CompilerParams(
            dimension_semantics=("arbitrary", "arbitrary")),
    )(pc, intensity, sn, mki, cluster_mean,
      w1, _col(b1), w2, _col(b2), w3, _col(b3))

    (v1, c1), (v2, c2) = params['pn2']
    second, node_a_feat = pl.pallas_call(
        _pn2_body,
        grid=grid1,
        in_specs=[
            pl.BlockSpec((1, _CA // 2, _CH), lambda b, j: (b, 0, j)),
            pl.BlockSpec((1, _CH, 3), lambda b, j: (b, j, 0)),
            pl.BlockSpec((1, _MA, _CA // 2), lambda b, j: (b, 0, 0)),
            wspec(v1.shape), wspec((1, v1.shape[0])),
            wspec(v2.shape), wspec((1, v2.shape[0])),
        ],
        out_specs=[
            pl.BlockSpec((1, _CA, _CH), lambda b, j: (b, 0, j)),
            pl.BlockSpec((1, _CA, _MA), lambda b, j: (b, 0, 0)),
        ],
        out_shape=[
            jax.ShapeDtypeStruct((_B, _CA, _N), f32),
            jax.ShapeDtypeStruct((_B, _CA, _MA), f32),
        ],
        scratch_shapes=[pltpu.VMEM((_MA, _CA), f32),
                        pltpu.VMEM((_CH, _CA), f32)],
        compiler_params=pltpu.CompilerParams(
            dimension_semantics=("arbitrary", "arbitrary")),
    )(first, mki, fmax_pm, v1, c1.reshape(1, -1), v2, c2.reshape(1, -1))

    (k1w1, k1b1), (k1w2, k1b2) = params['knn1']
    (k2w1, k2b1), (k2w2, k2b2) = params['knn2']
    (fw1, fb1), (fw2, fb2) = params['pnf']
    node_b_feat, global_feature = pl.pallas_call(
        _knn_body,
        grid=(_B,),
        in_specs=[
            pl.BlockSpec((1, 3, _MB), lambda b: (b, 0, 0)),
            pl.BlockSpec((1, 3, _MA), lambda b: (b, 0, 0)),
            pl.BlockSpec((1, _CA, _MA), lambda b: (b, 0, 0)),
            pl.BlockSpec(k1w1.shape, lambda b: (0, 0)),
            pl.BlockSpec((k1w1.shape[0], 1), lambda b: (0, 0)),
            pl.BlockSpec(k1w2.shape, lambda b: (0, 0)),
            pl.BlockSpec((k1w2.shape[0], 1), lambda b: (0, 0)),
            pl.BlockSpec(k2w1.shape, lambda b: (0, 0)),
            pl.BlockSpec((k2w1.shape[0], 1), lambda b: (0, 0)),
            pl.BlockSpec(k2w2.shape, lambda b: (0, 0)),
            pl.BlockSpec((k2w2.shape[0], 1), lambda b: (0, 0)),
            pl.BlockSpec(fw1.shape, lambda b: (0, 0)),
            pl.BlockSpec((fw1.shape[0], 1), lambda b: (0, 0)),
            pl.BlockSpec(fw2.shape, lambda b: (0, 0)),
            pl.BlockSpec((fw2.shape[0], 1), lambda b: (0, 0)),
        ],
        out_specs=[
            pl.BlockSpec((1, _CB, _MB), lambda b: (b, 0, 0)),
            pl.BlockSpec((1, _CG, 1), lambda b: (b, 0, 0)),
        ],
        out_shape=[
            jax.ShapeDtypeStruct((_B, _CB, _MB), f32),
            jax.ShapeDtypeStruct((_B, _CG, 1), f32),
        ],
        compiler_params=pltpu.CompilerParams(
            dimension_semantics=("arbitrary",)),
    )(node_b, cluster_mean, node_a_feat,
      k1w1, _col(k1b1), k1w2, _col(k1b2),
      k2w1, _col(k2b1), k2w2, _col(k2b2),
      fw1, _col(fb1), fw2, _col(fb2))

    return (pc_centers, cluster_mean, mki, first, second,
            node_a_feat, node_b_feat, global_feature)
